# Initial kernel scaffold; baseline (speedup 1.0000x reference)
import jax
import jax.numpy as jnp
from jax.experimental import pallas as pl


def kernel(y_pred, y_true):
    raise NotImplementedError("write your pallas kernel here")


# speedup vs baseline: 6.2047x; 6.2047x over previous
import functools

import numpy as np
import jax
import jax.numpy as jnp
from jax import lax
from jax.experimental import pallas as pl
from jax.experimental.pallas import tpu as pltpu
from jax.experimental.pallas import tpu_sc as plsc

_EPS = np.float32(1e-06)
_PAD = np.float32(-1.0)
_INF = np.float32(np.inf)
_NEG_INF = np.float32(-np.inf)
_LN2 = np.float32(0.6931471805599453)
_SQRT2 = np.float32(1.4142135623730951)
_C3 = np.float32(1.0 / 3.0)
_C5 = np.float32(1.0 / 5.0)
_C7 = np.float32(1.0 / 7.0)

_NC, _NS, _LANES = 2, 16, 16
_NW = _NC * _NS
_NV = 16


def _swlog(x):
    bits = plsc.bitcast(x, jnp.int32)
    e = ((bits >> 23) & 0xFF) - 127
    mant = plsc.bitcast((bits & 0x007FFFFF) | 0x3F800000, jnp.float32)
    big = mant > _SQRT2
    mant = jnp.where(big, mant * np.float32(0.5), mant)
    ef = jnp.where(big, e + 1, e).astype(jnp.float32)
    z = (mant - np.float32(1.0)) / (mant + np.float32(1.0))
    w = z * z
    poly = np.float32(2.0) * z * (np.float32(1.0) + w * (_C3 + w * (_C5 + w * _C7)))
    return ef * _LN2 + poly


def _bitonic_merge(ks, vs):
    n = len(ks)
    a = n // 2
    ks = ks[:a] + [lax.rev(ks[n - 1 - i], (0,)) for i in range(a)]
    vs = vs[:a] + [lax.rev(vs[n - 1 - i], (0,)) for i in range(a)]
    d = a
    while d >= 1:
        for i in range(n):
            if (i // d) % 2 == 0:
                j = i + d
                cond = ks[i] <= ks[j]
                klo = jnp.minimum(ks[i], ks[j])
                khi = jnp.maximum(ks[i], ks[j])
                vlo = jnp.where(cond, vs[i], vs[j])
                vhi = jnp.where(cond, vs[j], vs[i])
                ks[i], vs[i], ks[j], vs[j] = klo, vlo, khi, vhi
        d //= 2
    sorted_pairs = [plsc.sort_key_val(ks[i], vs[i]) for i in range(n)]
    return [p[0] for p in sorted_pairs], [p[1] for p in sorted_pairs]


def _sort_row(ks, vs):
    nv = len(ks)
    pairs = [plsc.sort_key_val(ks[i], vs[i]) for i in range(nv)]
    ks = [p[0] for p in pairs]
    vs = [p[1] for p in pairs]
    run = 1
    while run < nv:
        nks, nvs = [], []
        for s in range(0, nv, 2 * run):
            mk, mv = _bitonic_merge(ks[s:s + 2 * run], vs[s:s + 2 * run])
            nks += mk
            nvs += mv
        ks, vs = nks, nvs
        run *= 2
    return ks, vs


def _make_sc_call(B, L, chunk):
    rows_per_w = B // _NW
    n_chunks = rows_per_w // chunk
    nfull = L // _LANES
    tail = L - nfull * _LANES
    nreal = nfull + (1 if tail else 0)
    mesh = plsc.VectorSubcoreMesh(
        core_axis_name="c", subcore_axis_name="s",
        num_cores=_NC, num_subcores=_NS)

    @functools.partial(
        pl.kernel,
        out_type=jax.ShapeDtypeStruct((_NW * _LANES,), jnp.float32),
        mesh=mesh,
        compiler_params=pltpu.CompilerParams(needs_layout_passes=False),
        scratch_types=[
            pltpu.VMEM((chunk, L), jnp.float32),
            pltpu.VMEM((chunk, L), jnp.float32),
            pltpu.VMEM((_LANES,), jnp.float32),
        ],
    )
    def sc_kernel(yp_hbm, yt_hbm, out_hbm, pbuf, tbuf, outv):
        wid = lax.axis_index("s") * _NC + lax.axis_index("c")
        base = wid * rows_per_w
        lane = lax.iota(jnp.int32, 16)
        dup = lane < (_LANES - tail)

        def chunk_body(ci, acc):
            row0 = base + ci * chunk
            pltpu.sync_copy(yt_hbm.at[pl.ds(row0, chunk)], tbuf)
            pltpu.sync_copy(yp_hbm.at[pl.ds(row0, chunk)], pbuf)

            def row_body(r, acc2):
                ts, pms = [], []
                for i in range(nreal):
                    off = i * _LANES if i < nfull else L - _LANES
                    t = tbuf[r, pl.ds(off, _LANES)]
                    p = pbuf[r, pl.ds(off, _LANES)]
                    if i == nfull:
                        t = jnp.where(dup, _INF, t)
                    pm = jnp.where(t == _PAD, _NEG_INF, p)
                    if i == nfull:
                        pm = jnp.where(dup, _NEG_INF, pm)
                    ts.append(t)
                    pms.append(pm)
                mv = pms[0]
                for i in range(1, nreal):
                    mv = jnp.maximum(mv, pms[i])
                m = jnp.max(mv)
                es, lin = [], None
                for i in range(nreal):
                    gone = pms[i] == _NEG_INF
                    es.append(jnp.where(gone, np.float32(0.0), jnp.exp(pms[i] - m)))
                    term = jnp.where(gone, np.float32(0.0), pms[i] - m)
                    lin = term if lin is None else lin + term
                keys = ts + [jnp.full((_LANES,), _INF, jnp.float32)] * (_NV - nreal)
                vals = es + [jnp.zeros((_LANES,), jnp.float32)] * (_NV - nreal)
                sk, sv = _sort_row(keys, vals)
                carry = np.float32(0.0)
                logsum = jnp.zeros((_LANES,), jnp.float32)
                for i in range(_NV):
                    cs = plsc.cumsum(sv[i]) + carry
                    carry = carry + jnp.sum(sv[i])
                    lg = _swlog(cs + _EPS)
                    valid = (sk[i] != _PAD) & (sk[i] != _INF)
                    logsum = logsum + jnp.where(valid, lg, np.float32(0.0))
                return acc2 + (logsum - lin)

            return lax.fori_loop(0, chunk, row_body, acc)

        acc = lax.fori_loop(0, n_chunks, chunk_body,
                            jnp.zeros((_LANES,), jnp.float32))
        outv[...] = acc
        pltpu.sync_copy(outv, out_hbm.at[pl.ds(wid * _LANES, _LANES)])

    return sc_kernel


def kernel(y_pred, y_true):
    B, L = y_pred.shape
    sc_call = _make_sc_call(B, L, chunk=64)
    partials = sc_call(y_pred, y_true)
    return jnp.sum(partials) / np.float32(B)


# speedup vs baseline: 7.1894x; 1.1587x over previous
import functools

import numpy as np
import jax
import jax.numpy as jnp
from jax import lax
from jax.experimental import pallas as pl
from jax.experimental.pallas import tpu as pltpu
from jax.experimental.pallas import tpu_sc as plsc

_EPS = np.float32(1e-06)
_PAD = np.float32(-1.0)
_INF = np.float32(np.inf)
_NEG_INF = np.float32(-np.inf)
_LN2 = np.float32(0.6931471805599453)
_SQRT2 = np.float32(1.4142135623730951)
_C3 = np.float32(1.0 / 3.0)
_C5 = np.float32(1.0 / 5.0)
_C7 = np.float32(1.0 / 7.0)

_NC, _NS, _LANES = 2, 16, 16
_NW = _NC * _NS
_NV = 16


def _swlog(x):
    bits = plsc.bitcast(x, jnp.int32)
    e = ((bits >> 23) & 0xFF) - 127
    mant = plsc.bitcast((bits & 0x007FFFFF) | 0x3F800000, jnp.float32)
    big = mant > _SQRT2
    mant = jnp.where(big, mant * np.float32(0.5), mant)
    ef = jnp.where(big, e + 1, e).astype(jnp.float32)
    z = (mant - np.float32(1.0)) / (mant + np.float32(1.0))
    w = z * z
    poly = np.float32(2.0) * z * (np.float32(1.0) + w * (_C3 + w * (_C5 + w * _C7)))
    return ef * _LN2 + poly


class _Vr:
    __slots__ = ("k", "v", "isinf", "dirty")

    def __init__(self, k, v, isinf=False, dirty=True):
        self.k, self.v, self.isinf, self.dirty = k, v, isinf, dirty


def _vsort(x):
    if x.isinf or not x.dirty:
        return x
    k, v = plsc.sort_key_val(x.k, x.v)
    return _Vr(k, v, dirty=False)


def _vrev(x):
    if x.isinf:
        return x
    return _Vr(lax.rev(x.k, (0,)), lax.rev(x.v, (0,)), dirty=True)


def _vce(a, b):
    if b.isinf or (a.isinf and b.isinf):
        return a, b
    if a.isinf:
        return b, a
    cond = a.k <= b.k
    lo = _Vr(jnp.minimum(a.k, b.k), jnp.where(cond, a.v, b.v))
    hi = _Vr(jnp.maximum(a.k, b.k), jnp.where(cond, b.v, a.v))
    return lo, hi


def _bitonic_merge(xs):
    n = len(xs)
    a = n // 2
    xs = xs[:a] + [_vrev(xs[n - 1 - i]) for i in range(a)]
    d = a
    while d >= 1:
        for i in range(n):
            if (i // d) % 2 == 0:
                xs[i], xs[i + d] = _vce(xs[i], xs[i + d])
        d //= 2
    return [_vsort(x) for x in xs]


def _sort_row(xs):
    nv = len(xs)
    xs = [_vsort(x) for x in xs]
    run = 1
    while run < nv:
        nxs = []
        for s in range(0, nv, 2 * run):
            nxs += _bitonic_merge(xs[s:s + 2 * run])
        xs = nxs
        run *= 2
    return xs


def _make_sc_call(B, L, chunk):
    rows_per_w = B // _NW
    n_chunks = rows_per_w // chunk
    nfull = L // _LANES
    tail = L - nfull * _LANES
    nreal = nfull + (1 if tail else 0)
    mesh = plsc.VectorSubcoreMesh(
        core_axis_name="c", subcore_axis_name="s",
        num_cores=_NC, num_subcores=_NS)

    @functools.partial(
        pl.kernel,
        out_type=jax.ShapeDtypeStruct((_NW * _LANES,), jnp.float32),
        mesh=mesh,
        compiler_params=pltpu.CompilerParams(needs_layout_passes=False),
        scratch_types=[
            pltpu.VMEM((chunk, L), jnp.float32),
            pltpu.VMEM((chunk, L), jnp.float32),
            pltpu.VMEM((_LANES,), jnp.float32),
        ],
    )
    def sc_kernel(yp_hbm, yt_hbm, out_hbm, pbuf, tbuf, outv):
        wid = lax.axis_index("s") * _NC + lax.axis_index("c")
        base = wid * rows_per_w
        lane = lax.iota(jnp.int32, 16)
        dup = lane < (_LANES - tail)

        def chunk_body(ci, acc):
            row0 = base + ci * chunk
            pltpu.sync_copy(yt_hbm.at[pl.ds(row0, chunk)], tbuf)
            pltpu.sync_copy(yp_hbm.at[pl.ds(row0, chunk)], pbuf)

            def row_body(r, acc2):
                ts, pms = [], []
                for i in range(nreal):
                    off = i * _LANES if i < nfull else L - _LANES
                    t = tbuf[r, pl.ds(off, _LANES)]
                    p = pbuf[r, pl.ds(off, _LANES)]
                    if i == nfull:
                        t = jnp.where(dup, _INF, t)
                    pm = jnp.where(t == _PAD, _NEG_INF, p)
                    if i == nfull:
                        pm = jnp.where(dup, _NEG_INF, pm)
                    ts.append(t)
                    pms.append(pm)
                mv = pms[0]
                for i in range(1, nreal):
                    mv = jnp.maximum(mv, pms[i])
                m = jnp.max(mv)
                es, lin = [], None
                for i in range(nreal):
                    gone = pms[i] == _NEG_INF
                    es.append(jnp.where(gone, np.float32(0.0), jnp.exp(pms[i] - m)))
                    term = jnp.where(gone, np.float32(0.0), pms[i] - m)
                    lin = term if lin is None else lin + term
                pad_k = jnp.full((_LANES,), _INF, jnp.float32)
                pad_v = jnp.zeros((_LANES,), jnp.float32)
                xs = [_Vr(ts[i], es[i]) for i in range(nreal)]
                xs += [_Vr(pad_k, pad_v, isinf=True, dirty=False)
                       for _ in range(_NV - nreal)]
                xs = _sort_row(xs)
                carry = np.float32(0.0)
                logsum = jnp.zeros((_LANES,), jnp.float32)
                for x in xs:
                    if x.isinf:
                        continue
                    cs = plsc.cumsum(x.v) + carry
                    carry = carry + jnp.sum(x.v)
                    lg = _swlog(cs + _EPS)
                    valid = (x.k != _PAD) & (x.k != _INF)
                    logsum = logsum + jnp.where(valid, lg, np.float32(0.0))
                return acc2 + (logsum - lin)

            return lax.fori_loop(0, chunk, row_body, acc)

        acc = lax.fori_loop(0, n_chunks, chunk_body,
                            jnp.zeros((_LANES,), jnp.float32))
        outv[...] = acc
        pltpu.sync_copy(outv, out_hbm.at[pl.ds(wid * _LANES, _LANES)])

    return sc_kernel


def kernel(y_pred, y_true):
    B, L = y_pred.shape
    sc_call = _make_sc_call(B, L, chunk=64)
    partials = sc_call(y_pred, y_true)
    return jnp.sum(partials) / np.float32(B)


# speedup vs baseline: 7.8509x; 1.0920x over previous
import functools

import numpy as np
import jax
import jax.numpy as jnp
from jax import lax
from jax.experimental import pallas as pl
from jax.experimental.pallas import tpu as pltpu
from jax.experimental.pallas import tpu_sc as plsc

_EPS = np.float32(1e-06)
_PAD = np.float32(-1.0)
_NEG_INF = np.float32(-np.inf)
_LN2 = np.float32(0.6931471805599453)
_C3 = np.float32(1.0 / 3.0)
_C5 = np.float32(1.0 / 5.0)
_C7 = np.float32(1.0 / 7.0)
_C9 = np.float32(1.0 / 9.0)

_NC, _NS, _LANES = 2, 16, 16
_NW = _NC * _NS
_NV = 16

_PAD_MAX = np.int32(0x7FFFFFFF)


def _mono_i32(bits):
    return bits ^ ((bits >> 31) & np.int32(0x7FFFFFFF))


_PADBUCKET = np.int32(np.uint32(
    (np.uint32(0xBF800000 ^ 0x7FFFFFFF) & np.uint32(0xFFFFFF00))).astype(np.int64) - (1 << 32))
_ALLBUCKET = np.int32(np.uint32(0x7FFFFF00))
_HI24 = np.int32(np.uint32(0xFFFFFF00).astype(np.int64) - (1 << 32))
_LO8 = np.int32(0xFF)


def _swlog(x):
    bits = plsc.bitcast(x, jnp.int32)
    e = ((bits >> 23) & 0xFF) - 127
    mant = plsc.bitcast((bits & 0x007FFFFF) | 0x3F800000, jnp.float32)
    z = (mant - np.float32(1.0)) / (mant + np.float32(1.0))
    w = z * z
    poly = (np.float32(2.0) * z *
            (np.float32(1.0) + w * (_C3 + w * (_C5 + w * (_C7 + w * _C9)))))
    return e.astype(jnp.float32) * _LN2 + poly


class _Vr:
    __slots__ = ("k", "ispad", "dirty")

    def __init__(self, k, ispad=False, dirty=True):
        self.k, self.ispad, self.dirty = k, ispad, dirty


def _vsort(x):
    if x.ispad or not x.dirty:
        return x
    return _Vr(jnp.sort(x.k), dirty=False)


def _vrev(x):
    if x.ispad:
        return x
    return _Vr(lax.rev(x.k, (0,)), dirty=True)


def _vce(a, b):
    if b.ispad:
        return a, b
    if a.ispad:
        return b, a
    return (_Vr(jnp.minimum(a.k, b.k)), _Vr(jnp.maximum(a.k, b.k)))


def _bitonic_merge(xs):
    n = len(xs)
    a = n // 2
    xs = xs[:a] + [_vrev(xs[n - 1 - i]) for i in range(a)]
    d = a
    while d >= 1:
        for i in range(n):
            if (i // d) % 2 == 0:
                xs[i], xs[i + d] = _vce(xs[i], xs[i + d])
        d //= 2
    return [_vsort(x) for x in xs]


def _sort_row(xs):
    nv = len(xs)
    xs = [_vsort(x) for x in xs]
    run = 1
    while run < nv:
        nxs = []
        for s in range(0, nv, 2 * run):
            nxs += _bitonic_merge(xs[s:s + 2 * run])
        xs = nxs
        run *= 2
    return xs


def _make_sc_call(B, L, chunk):
    rows_per_w = B // _NW
    n_chunks = rows_per_w // chunk
    nfull = L // _LANES
    tail = L - nfull * _LANES
    nreal = nfull + (1 if tail else 0)
    ebuf_sz = 256
    mesh = plsc.VectorSubcoreMesh(
        core_axis_name="c", subcore_axis_name="s",
        num_cores=_NC, num_subcores=_NS)

    @functools.partial(
        pl.kernel,
        out_type=jax.ShapeDtypeStruct((_NW * _LANES,), jnp.float32),
        mesh=mesh,
        compiler_params=pltpu.CompilerParams(needs_layout_passes=False),
        scratch_types=[
            pltpu.VMEM((chunk, L), jnp.float32),
            pltpu.VMEM((chunk, L), jnp.float32),
            pltpu.VMEM((ebuf_sz,), jnp.float32),
            pltpu.VMEM((_LANES,), jnp.float32),
        ],
    )
    def sc_kernel(yp_hbm, yt_hbm, out_hbm, pbuf, tbuf, ebuf, outv):
        wid = lax.axis_index("s") * _NC + lax.axis_index("c")
        base = wid * rows_per_w
        lane = lax.iota(jnp.int32, 16)
        dup = lane < (_LANES - tail)
        for off in range(nfull * _LANES, ebuf_sz, _LANES):
            ebuf[pl.ds(off, _LANES)] = jnp.zeros((_LANES,), jnp.float32)

        def chunk_body(ci, acc):
            row0 = base + ci * chunk
            pltpu.sync_copy(yt_hbm.at[pl.ds(row0, chunk)], tbuf)
            pltpu.sync_copy(yp_hbm.at[pl.ds(row0, chunk)], pbuf)

            def row_body(r, acc2):
                ts, pms = [], []
                for i in range(nreal):
                    off = i * _LANES if i < nfull else L - _LANES
                    t = tbuf[r, pl.ds(off, _LANES)]
                    p = pbuf[r, pl.ds(off, _LANES)]
                    pm = jnp.where(t == _PAD, _NEG_INF, p)
                    if i == nfull:
                        pm = jnp.where(dup, _NEG_INF, pm)
                    ts.append(t)
                    pms.append(pm)
                mv = pms[0]
                for i in range(1, nreal):
                    mv = jnp.maximum(mv, pms[i])
                m = jnp.max(mv)
                lin = None
                evs = []
                for i in range(nreal):
                    gone = pms[i] == _NEG_INF
                    evs.append(jnp.where(gone, np.float32(0.0),
                                         jnp.exp(pms[i] - m)))
                    term = jnp.where(gone, np.float32(0.0), pms[i] - m)
                    lin = term if lin is None else lin + term
                order = ([nfull] if tail else []) + list(range(nfull))
                for i in order:
                    off = i * _LANES if i < nfull else L - _LANES
                    ebuf[pl.ds(off, _LANES)] = evs[i]
                xs = []
                for i in range(nreal):
                    off = i * _LANES if i < nfull else L - _LANES
                    key = _mono_i32(plsc.bitcast(ts[i], jnp.int32))
                    key = (key & _HI24) | (lane + np.int32(off))
                    if i == nfull:
                        key = jnp.where(dup, _PAD_MAX, key)
                    xs.append(_Vr(key))
                pad_k = jnp.full((_LANES,), _PAD_MAX, jnp.int32)
                xs += [_Vr(pad_k, ispad=True, dirty=False)
                       for _ in range(_NV - nreal)]
                xs = _sort_row(xs)
                carry = np.float32(0.0)
                logsum = jnp.zeros((_LANES,), jnp.float32)
                for x in xs:
                    if x.ispad:
                        continue
                    es = plsc.load_gather(ebuf, [x.k & _LO8])
                    cs = plsc.cumsum(es) + carry
                    carry = carry + jnp.sum(es)
                    lg = _swlog(cs + _EPS)
                    h = x.k & _HI24
                    valid = (h != _PADBUCKET) & (h != _ALLBUCKET)
                    logsum = logsum + jnp.where(valid, lg, np.float32(0.0))
                return acc2 + (logsum - lin)

            return lax.fori_loop(0, chunk, row_body, acc)

        acc = lax.fori_loop(0, n_chunks, chunk_body,
                            jnp.zeros((_LANES,), jnp.float32))
        outv[...] = acc
        pltpu.sync_copy(outv, out_hbm.at[pl.ds(wid * _LANES, _LANES)])

    return sc_kernel


def kernel(y_pred, y_true):
    B, L = y_pred.shape
    sc_call = _make_sc_call(B, L, chunk=64)
    partials = sc_call(y_pred, y_true)
    return jnp.sum(partials) / np.float32(B)


# speedup vs baseline: 8.3483x; 1.0634x over previous
import functools

import numpy as np
import jax
import jax.numpy as jnp
from jax import lax
from jax.experimental import pallas as pl
from jax.experimental.pallas import tpu as pltpu
from jax.experimental.pallas import tpu_sc as plsc

_EPS = np.float32(1e-06)
_PAD = np.float32(-1.0)
_NEG_INF = np.float32(-np.inf)
_LN2 = np.float32(0.6931471805599453)
_C3 = np.float32(1.0 / 3.0)
_C5 = np.float32(1.0 / 5.0)
_C7 = np.float32(1.0 / 7.0)
_C9 = np.float32(1.0 / 9.0)

_NC, _NS, _LANES = 2, 16, 16
_NW = _NC * _NS
_NV = 16

_PAD_KEY = np.float32(np.inf)
_PADBITS = np.int32(np.int64(0xBF800000) - (1 << 32))
_INFBITS = np.int32(0x7F800000)
_HI24 = np.int32(np.int64(0xFFFFFF00) - (1 << 32))
_LO8 = np.int32(0xFF)


def _swlog(x):
    bits = plsc.bitcast(x, jnp.int32)
    e = ((bits >> 23) & 0xFF) - 127
    mant = plsc.bitcast((bits & 0x007FFFFF) | 0x3F800000, jnp.float32)
    z = (mant - np.float32(1.0)) / (mant + np.float32(1.0))
    w = z * z
    poly = (np.float32(2.0) * z *
            (np.float32(1.0) + w * (_C3 + w * (_C5 + w * (_C7 + w * _C9)))))
    return e.astype(jnp.float32) * _LN2 + poly


class _Vr:
    __slots__ = ("k", "ispad", "dirty")

    def __init__(self, k, ispad=False, dirty=True):
        self.k, self.ispad, self.dirty = k, ispad, dirty


def _vsort(x):
    if x.ispad or not x.dirty:
        return x
    return _Vr(jnp.sort(x.k), dirty=False)


def _vrev(x):
    if x.ispad:
        return x
    return _Vr(lax.rev(x.k, (0,)), dirty=True)


def _vce(a, b):
    if b.ispad:
        return a, b
    if a.ispad:
        return b, a
    return (_Vr(jnp.minimum(a.k, b.k)), _Vr(jnp.maximum(a.k, b.k)))


def _bitonic_merge(xs):
    n = len(xs)
    a = n // 2
    xs = xs[:a] + [_vrev(xs[n - 1 - i]) for i in range(a)]
    d = a
    while d >= 1:
        for i in range(n):
            if (i // d) % 2 == 0:
                xs[i], xs[i + d] = _vce(xs[i], xs[i + d])
        d //= 2
    return [_vsort(x) for x in xs]


def _sort_row(xs):
    nv = len(xs)
    xs = [_vsort(x) for x in xs]
    run = 1
    while run < nv:
        nxs = []
        for s in range(0, nv, 2 * run):
            nxs += _bitonic_merge(xs[s:s + 2 * run])
        xs = nxs
        run *= 2
    return xs


def _make_sc_call(B, L, chunk):
    rows_per_w = B // _NW
    n_chunks = rows_per_w // chunk
    nfull = L // _LANES
    tail = L - nfull * _LANES
    nreal = nfull + (1 if tail else 0)
    ebuf_sz = 256
    mesh = plsc.VectorSubcoreMesh(
        core_axis_name="c", subcore_axis_name="s",
        num_cores=_NC, num_subcores=_NS)

    @functools.partial(
        pl.kernel,
        out_type=jax.ShapeDtypeStruct((_NW * _LANES,), jnp.float32),
        mesh=mesh,
        compiler_params=pltpu.CompilerParams(needs_layout_passes=False),
        scratch_types=[
            pltpu.VMEM((chunk, L), jnp.float32),
            pltpu.VMEM((chunk, L), jnp.float32),
            pltpu.VMEM((ebuf_sz,), jnp.float32),
            pltpu.VMEM((_LANES,), jnp.float32),
        ],
    )
    def sc_kernel(yp_hbm, yt_hbm, out_hbm, pbuf, tbuf, ebuf, outv):
        wid = lax.axis_index("s") * _NC + lax.axis_index("c")
        base = wid * rows_per_w
        lane = lax.iota(jnp.int32, 16)
        dup = lane < (_LANES - tail)
        for off in range(nfull * _LANES, ebuf_sz, _LANES):
            ebuf[pl.ds(off, _LANES)] = jnp.zeros((_LANES,), jnp.float32)

        def chunk_body(ci, acc):
            row0 = base + ci * chunk
            pltpu.sync_copy(yt_hbm.at[pl.ds(row0, chunk)], tbuf)
            pltpu.sync_copy(yp_hbm.at[pl.ds(row0, chunk)], pbuf)

            def row_body(r, acc2):
                ts, pms = [], []
                for i in range(nreal):
                    off = i * _LANES if i < nfull else L - _LANES
                    t = tbuf[r, pl.ds(off, _LANES)]
                    p = pbuf[r, pl.ds(off, _LANES)]
                    pm = jnp.where(t == _PAD, _NEG_INF, p)
                    if i == nfull:
                        pm = jnp.where(dup, _NEG_INF, pm)
                    ts.append(t)
                    pms.append(pm)
                mv = pms[0]
                for i in range(1, nreal):
                    mv = jnp.maximum(mv, pms[i])
                m = jnp.max(mv)
                lin = None
                evs = []
                for i in range(nreal):
                    gone = pms[i] == _NEG_INF
                    evs.append(jnp.where(gone, np.float32(0.0),
                                         jnp.exp(pms[i] - m)))
                    term = jnp.where(gone, np.float32(0.0), pms[i] - m)
                    lin = term if lin is None else lin + term
                order = ([nfull] if tail else []) + list(range(nfull))
                for i in order:
                    off = i * _LANES if i < nfull else L - _LANES
                    ebuf[pl.ds(off, _LANES)] = evs[i]
                xs = []
                for i in range(nreal):
                    off = i * _LANES if i < nfull else L - _LANES
                    kb = (plsc.bitcast(ts[i], jnp.int32) & _HI24) \
                        | (lane + np.int32(off))
                    key = plsc.bitcast(kb, jnp.float32)
                    if i == nfull:
                        key = jnp.where(dup, _PAD_KEY, key)
                    xs.append(_Vr(key))
                pad_k = jnp.full((_LANES,), _PAD_KEY, jnp.float32)
                xs += [_Vr(pad_k, ispad=True, dirty=False)
                       for _ in range(_NV - nreal)]
                xs = _sort_row(xs)
                carry = np.float32(0.0)
                logsum = jnp.zeros((_LANES,), jnp.float32)
                nlast = nreal - 1
                for pos, x in enumerate(xs):
                    if x.ispad:
                        continue
                    kb = plsc.bitcast(x.k, jnp.int32)
                    es = plsc.load_gather(ebuf, [kb & _LO8])
                    h = kb & _HI24
                    if pos == nlast:
                        isinf = h == _INFBITS
                        es = jnp.where(isinf, np.float32(0.0), es)
                        valid = (h != _PADBITS) & (~isinf)
                    else:
                        valid = h != _PADBITS
                    cs = plsc.cumsum(es) + carry
                    carry = carry + jnp.sum(es)
                    lg = _swlog(cs + _EPS)
                    logsum = logsum + jnp.where(valid, lg, np.float32(0.0))
                return acc2 + (logsum - lin)

            return lax.fori_loop(0, chunk, row_body, acc)

        acc = lax.fori_loop(0, n_chunks, chunk_body,
                            jnp.zeros((_LANES,), jnp.float32))
        outv[...] = acc
        pltpu.sync_copy(outv, out_hbm.at[pl.ds(wid * _LANES, _LANES)])

    return sc_kernel


def kernel(y_pred, y_true):
    B, L = y_pred.shape
    sc_call = _make_sc_call(B, L, chunk=64)
    partials = sc_call(y_pred, y_true)
    return jnp.sum(partials) / np.float32(B)


# speedup vs baseline: 8.4272x; 1.0095x over previous
import functools

import numpy as np
import jax
import jax.numpy as jnp
from jax import lax
from jax.experimental import pallas as pl
from jax.experimental.pallas import tpu as pltpu
from jax.experimental.pallas import tpu_sc as plsc

_EPS = np.float32(1e-06)
_PAD = np.float32(-1.0)
_NEG_INF = np.float32(-np.inf)
_LN2 = np.float32(0.6931471805599453)
_C3 = np.float32(1.0 / 3.0)
_C5 = np.float32(1.0 / 5.0)
_C7 = np.float32(1.0 / 7.0)
_C9 = np.float32(1.0 / 9.0)

_NC, _NS, _LANES = 2, 16, 16
_NW = _NC * _NS
_NV = 16

_PAD_KEY = np.float32(np.inf)
_PADBITS = np.int32(np.int64(0xBF800000) - (1 << 32))
_INFBITS = np.int32(0x7F800000)
_HI24 = np.int32(np.int64(0xFFFFFF00) - (1 << 32))
_LO8 = np.int32(0xFF)


def _swlog(x):
    bits = plsc.bitcast(x, jnp.int32)
    e = ((bits >> 23) & 0xFF) - 127
    mant = plsc.bitcast((bits & 0x007FFFFF) | 0x3F800000, jnp.float32)
    z = (mant - np.float32(1.0)) / (mant + np.float32(1.0))
    w = z * z
    poly = (np.float32(2.0) * z *
            (np.float32(1.0) + w * (_C3 + w * (_C5 + w * (_C7 + w * _C9)))))
    return e.astype(jnp.float32) * _LN2 + poly


class _Vr:
    __slots__ = ("k", "ispad", "dirty")

    def __init__(self, k, ispad=False, dirty=True):
        self.k, self.ispad, self.dirty = k, ispad, dirty


def _vsort(x):
    if x.ispad or not x.dirty:
        return x
    return _Vr(jnp.sort(x.k), dirty=False)


def _vrev(x):
    if x.ispad:
        return x
    return _Vr(lax.rev(x.k, (0,)), dirty=True)


def _vce(a, b):
    if b.ispad:
        return a, b
    if a.ispad:
        return b, a
    return (_Vr(jnp.minimum(a.k, b.k)), _Vr(jnp.maximum(a.k, b.k)))


def _bitonic_merge(xs):
    n = len(xs)
    a = n // 2
    xs = xs[:a] + [_vrev(xs[n - 1 - i]) for i in range(a)]
    d = a
    while d >= 1:
        for i in range(n):
            if (i // d) % 2 == 0:
                xs[i], xs[i + d] = _vce(xs[i], xs[i + d])
        d //= 2
    return [_vsort(x) for x in xs]


def _sort_row(xs):
    nv = len(xs)
    xs = [_vsort(x) for x in xs]
    run = 1
    while run < nv:
        nxs = []
        for s in range(0, nv, 2 * run):
            nxs += _bitonic_merge(xs[s:s + 2 * run])
        xs = nxs
        run *= 2
    return xs


def _make_sc_call(B, L, chunk):
    rows_per_w = B // _NW
    n_chunks = rows_per_w // chunk
    nfull = L // _LANES
    tail = L - nfull * _LANES
    nreal = nfull + (1 if tail else 0)
    ebuf_sz = 256
    mesh = plsc.VectorSubcoreMesh(
        core_axis_name="c", subcore_axis_name="s",
        num_cores=_NC, num_subcores=_NS)

    @functools.partial(
        pl.kernel,
        out_type=jax.ShapeDtypeStruct((_NW * _LANES,), jnp.float32),
        mesh=mesh,
        compiler_params=pltpu.CompilerParams(needs_layout_passes=False),
        scratch_types=[
            pltpu.VMEM((chunk, L), jnp.float32),
            pltpu.VMEM((chunk, L), jnp.float32),
            pltpu.VMEM((ebuf_sz,), jnp.float32),
            pltpu.VMEM((_LANES,), jnp.float32),
        ],
    )
    def sc_kernel(yp_hbm, yt_hbm, out_hbm, pbuf, tbuf, ebuf, outv):
        wid = lax.axis_index("s") * _NC + lax.axis_index("c")
        base = wid * rows_per_w
        lane = lax.iota(jnp.int32, 16)
        dup = lane < (_LANES - tail)
        for off in range(nfull * _LANES, ebuf_sz, _LANES):
            ebuf[pl.ds(off, _LANES)] = jnp.zeros((_LANES,), jnp.float32)

        def chunk_body(ci, acc):
            row0 = base + ci * chunk
            pltpu.sync_copy(yt_hbm.at[pl.ds(row0, chunk)], tbuf)
            pltpu.sync_copy(yp_hbm.at[pl.ds(row0, chunk)], pbuf)

            def row_body(r, acc2):
                ts, pms = [], []
                for i in range(nreal):
                    off = i * _LANES if i < nfull else L - _LANES
                    t = tbuf[r, pl.ds(off, _LANES)]
                    p = pbuf[r, pl.ds(off, _LANES)]
                    pm = jnp.where(t == _PAD, _NEG_INF, p)
                    if i == nfull:
                        pm = jnp.where(dup, _NEG_INF, pm)
                    ts.append(t)
                    pms.append(pm)
                mv = pms[0]
                for i in range(1, nreal):
                    mv = jnp.maximum(mv, pms[i])
                m = jnp.max(mv)
                lin = None
                evs = []
                for i in range(nreal):
                    gone = pms[i] == _NEG_INF
                    evs.append(jnp.where(gone, np.float32(0.0),
                                         jnp.exp(pms[i] - m)))
                    term = jnp.where(gone, np.float32(0.0), pms[i] - m)
                    lin = term if lin is None else lin + term
                order = ([nfull] if tail else []) + list(range(nfull))
                for i in order:
                    off = i * _LANES if i < nfull else L - _LANES
                    ebuf[pl.ds(off, _LANES)] = evs[i]
                xs = []
                for i in range(nreal):
                    off = i * _LANES if i < nfull else L - _LANES
                    kb = (plsc.bitcast(ts[i], jnp.int32) & _HI24) \
                        | (lane + np.int32(off))
                    key = plsc.bitcast(kb, jnp.float32)
                    if i == nfull:
                        key = jnp.where(dup, _PAD_KEY, key)
                    xs.append(_Vr(key))
                pad_k = jnp.full((_LANES,), _PAD_KEY, jnp.float32)
                xs += [_Vr(pad_k, ispad=True, dirty=False)
                       for _ in range(_NV - nreal)]
                xs = _sort_row(xs)
                carry = np.float32(0.0)
                logsum = jnp.zeros((_LANES,), jnp.float32)
                nlast = nreal - 1
                for pos, x in enumerate(xs):
                    if x.ispad:
                        continue
                    kb = plsc.bitcast(x.k, jnp.int32)
                    es = plsc.load_gather(ebuf, [kb & _LO8])
                    h = kb & _HI24
                    if pos == nlast:
                        isinf = h == _INFBITS
                        es = jnp.where(isinf, np.float32(0.0), es)
                        valid = (h != _PADBITS) & (~isinf)
                    else:
                        valid = h != _PADBITS
                    cs = plsc.cumsum(es) + carry
                    carry = carry + jnp.sum(es)
                    lg = _swlog(cs + _EPS)
                    logsum = logsum + jnp.where(valid, lg, np.float32(0.0))
                return acc2 + (logsum - lin)

            return lax.fori_loop(0, chunk, row_body, acc, unroll=2)

        acc = lax.fori_loop(0, n_chunks, chunk_body,
                            jnp.zeros((_LANES,), jnp.float32))
        outv[...] = acc
        pltpu.sync_copy(outv, out_hbm.at[pl.ds(wid * _LANES, _LANES)])

    return sc_kernel


def kernel(y_pred, y_true):
    B, L = y_pred.shape
    sc_call = _make_sc_call(B, L, chunk=64)
    partials = sc_call(y_pred, y_true)
    return jnp.sum(partials) / np.float32(B)


# speedup vs baseline: 8.9296x; 1.0596x over previous
import functools

import numpy as np
import jax
import jax.numpy as jnp
from jax import lax
from jax.experimental import pallas as pl
from jax.experimental.pallas import tpu as pltpu
from jax.experimental.pallas import tpu_sc as plsc

_EPS = np.float32(1e-06)
_PAD = np.float32(-1.0)
_NEG_INF = np.float32(-np.inf)
_LN2 = np.float32(0.6931471805599453)
_C3 = np.float32(1.0 / 3.0)
_C5 = np.float32(1.0 / 5.0)
_C7 = np.float32(1.0 / 7.0)
_C9 = np.float32(1.0 / 9.0)

_NC, _NS, _LANES = 2, 16, 16
_NW = _NC * _NS
_NV = 16

_PAD_KEY = np.float32(np.inf)
_PADBITS = np.int32(np.int64(0xBF800000) - (1 << 32))
_INFBITS = np.int32(0x7F800000)
_HI24 = np.int32(np.int64(0xFFFFFF00) - (1 << 32))
_LO8 = np.int32(0xFF)


def _swlog(x):
    bits = plsc.bitcast(x, jnp.int32)
    e = ((bits >> 23) & 0xFF) - 127
    mant = plsc.bitcast((bits & 0x007FFFFF) | 0x3F800000, jnp.float32)
    z = (mant - np.float32(1.0)) / (mant + np.float32(1.0))
    w = z * z
    poly = (np.float32(2.0) * z *
            (np.float32(1.0) + w * (_C3 + w * (_C5 + w * (_C7 + w * _C9)))))
    return e.astype(jnp.float32) * _LN2 + poly


_TBASE = int(np.float32(1e-6).view(np.int32)) >> 15
_TLEN = 7168


def _log_tables():
    idx = np.arange(_TLEN, dtype=np.int64)
    bits0 = (idx + _TBASE) << 15
    x0 = bits0.astype(np.int64).astype(np.int32).view(np.float32).astype(np.float64)
    x1 = (bits0 + 32768).astype(np.int64).astype(np.int32).view(np.float32).astype(np.float64)
    t = np.log(x0)
    d = (np.log(x1) - t) / 32768.0
    return t.astype(np.float32), d.astype(np.float32)


class _Vr:
    __slots__ = ("k", "ispad", "dirty")

    def __init__(self, k, ispad=False, dirty=True):
        self.k, self.ispad, self.dirty = k, ispad, dirty


def _vsort(x):
    if x.ispad or not x.dirty:
        return x
    return _Vr(jnp.sort(x.k), dirty=False)


def _vrev(x):
    if x.ispad:
        return x
    return _Vr(lax.rev(x.k, (0,)), dirty=True)


def _vce(a, b):
    if b.ispad:
        return a, b
    if a.ispad:
        return b, a
    return (_Vr(jnp.minimum(a.k, b.k)), _Vr(jnp.maximum(a.k, b.k)))


def _bitonic_merge(xs):
    n = len(xs)
    a = n // 2
    xs = xs[:a] + [_vrev(xs[n - 1 - i]) for i in range(a)]
    d = a
    while d >= 1:
        for i in range(n):
            if (i // d) % 2 == 0:
                xs[i], xs[i + d] = _vce(xs[i], xs[i + d])
        d //= 2
    return [_vsort(x) for x in xs]


def _sort_row(xs):
    nv = len(xs)
    xs = [_vsort(x) for x in xs]
    run = 1
    while run < nv:
        nxs = []
        for s in range(0, nv, 2 * run):
            nxs += _bitonic_merge(xs[s:s + 2 * run])
        xs = nxs
        run *= 2
    return xs


def _make_sc_call(B, L, chunk):
    rows_per_w = B // _NW
    n_chunks = rows_per_w // chunk
    nfull = L // _LANES
    tail = L - nfull * _LANES
    nreal = nfull + (1 if tail else 0)
    ebuf_sz = 256
    mesh = plsc.VectorSubcoreMesh(
        core_axis_name="c", subcore_axis_name="s",
        num_cores=_NC, num_subcores=_NS)

    @functools.partial(
        pl.kernel,
        out_type=jax.ShapeDtypeStruct((_NW * _LANES,), jnp.float32),
        mesh=mesh,
        compiler_params=pltpu.CompilerParams(needs_layout_passes=False),
        scratch_types=[
            pltpu.VMEM((chunk, L), jnp.float32),
            pltpu.VMEM((chunk, L), jnp.float32),
            pltpu.VMEM((ebuf_sz,), jnp.float32),
            pltpu.VMEM((_TLEN,), jnp.float32),
            pltpu.VMEM((_TLEN,), jnp.float32),
            pltpu.VMEM((_LANES,), jnp.float32),
        ],
    )
    def sc_kernel(yp_hbm, yt_hbm, logt_hbm, logd_hbm, out_hbm,
                  pbuf, tbuf, ebuf, logt, logd, outv):
        wid = lax.axis_index("s") * _NC + lax.axis_index("c")
        base = wid * rows_per_w
        lane = lax.iota(jnp.int32, 16)
        dup = lane < (_LANES - tail)
        pltpu.sync_copy(logt_hbm, logt)
        pltpu.sync_copy(logd_hbm, logd)
        for off in range(nfull * _LANES, ebuf_sz, _LANES):
            ebuf[pl.ds(off, _LANES)] = jnp.zeros((_LANES,), jnp.float32)

        def chunk_body(ci, acc):
            row0 = base + ci * chunk
            pltpu.sync_copy(yt_hbm.at[pl.ds(row0, chunk)], tbuf)
            pltpu.sync_copy(yp_hbm.at[pl.ds(row0, chunk)], pbuf)

            def row_body(r, acc2):
                ts, pms = [], []
                for i in range(nreal):
                    off = i * _LANES if i < nfull else L - _LANES
                    t = tbuf[r, pl.ds(off, _LANES)]
                    p = pbuf[r, pl.ds(off, _LANES)]
                    pm = jnp.where(t == _PAD, _NEG_INF, p)
                    if i == nfull:
                        pm = jnp.where(dup, _NEG_INF, pm)
                    ts.append(t)
                    pms.append(pm)
                mv = pms[0]
                for i in range(1, nreal):
                    mv = jnp.maximum(mv, pms[i])
                m = jnp.max(mv)
                lin = None
                evs = []
                for i in range(nreal):
                    gone = pms[i] == _NEG_INF
                    evs.append(jnp.where(gone, np.float32(0.0),
                                         jnp.exp(pms[i] - m)))
                    term = jnp.where(gone, np.float32(0.0), pms[i] - m)
                    lin = term if lin is None else lin + term
                order = ([nfull] if tail else []) + list(range(nfull))
                for i in order:
                    off = i * _LANES if i < nfull else L - _LANES
                    ebuf[pl.ds(off, _LANES)] = evs[i]
                xs = []
                for i in range(nreal):
                    off = i * _LANES if i < nfull else L - _LANES
                    kb = (plsc.bitcast(ts[i], jnp.int32) & _HI24) \
                        | (lane + np.int32(off))
                    key = plsc.bitcast(kb, jnp.float32)
                    if i == nfull:
                        key = jnp.where(dup, _PAD_KEY, key)
                    xs.append(_Vr(key))
                pad_k = jnp.full((_LANES,), _PAD_KEY, jnp.float32)
                xs += [_Vr(pad_k, ispad=True, dirty=False)
                       for _ in range(_NV - nreal)]
                xs = _sort_row(xs)
                carry = np.float32(0.0)
                logsum = jnp.zeros((_LANES,), jnp.float32)
                nlast = nreal - 1
                for pos, x in enumerate(xs):
                    if x.ispad:
                        continue
                    kb = plsc.bitcast(x.k, jnp.int32)
                    es = plsc.load_gather(ebuf, [kb & _LO8])
                    h = kb & _HI24
                    if pos == nlast:
                        isinf = h == _INFBITS
                        es = jnp.where(isinf, np.float32(0.0), es)
                        valid = (h != _PADBITS) & (~isinf)
                    else:
                        valid = h != _PADBITS
                    cs = plsc.cumsum(es) + carry
                    carry = carry + jnp.sum(es)
                    lbits = plsc.bitcast(cs + _EPS, jnp.int32)
                    ti = (lbits >> 15) - np.int32(_TBASE)
                    fl = (lbits & np.int32(0x7FFF)).astype(jnp.float32)
                    lg = (plsc.load_gather(logt, [ti])
                          + fl * plsc.load_gather(logd, [ti]))
                    logsum = logsum + jnp.where(valid, lg, np.float32(0.0))
                return acc2 + (logsum - lin)

            return lax.fori_loop(0, chunk, row_body, acc, unroll=2)

        acc = lax.fori_loop(0, n_chunks, chunk_body,
                            jnp.zeros((_LANES,), jnp.float32))
        outv[...] = acc
        pltpu.sync_copy(outv, out_hbm.at[pl.ds(wid * _LANES, _LANES)])

    return sc_kernel


def kernel(y_pred, y_true):
    B, L = y_pred.shape
    sc_call = _make_sc_call(B, L, chunk=64)
    logt, logd = _log_tables()
    partials = sc_call(y_pred, y_true, jnp.asarray(logt), jnp.asarray(logd))
    return jnp.sum(partials) / np.float32(B)
